# baseline (device time: 385382 ns/iter reference)
import functools

import jax
import jax.numpy as jnp
from jax import lax
from jax.experimental import pallas as pl
from jax.experimental.pallas import tpu as pltpu

N_DEV = 4


def _neighbor_barrier(left, right):
    barrier_sem = pltpu.get_barrier_semaphore()
    for nbr in (left, right):
        pl.semaphore_signal(
            barrier_sem, inc=1,
            device_id=(nbr,), device_id_type=pl.DeviceIdType.MESH,
        )
    pl.semaphore_wait(barrier_sem, 2)


def _exit_barrier(left, right):
    @functools.partial(pl.run_scoped, sem=pltpu.SemaphoreType.REGULAR)
    def _(sem):
        for nbr in (left, right):
            pl.semaphore_signal(
                sem, inc=1,
                device_id=(nbr,), device_id_type=pl.DeviceIdType.MESH,
            )
        pl.semaphore_wait(sem, 2)


def _ring_allgather(x_shard):
    m, n = x_shard.shape

    def body(x_ref, out_ref, send_sems, recv_sems):
        me = lax.axis_index("i")
        left = lax.rem(me + N_DEV - 1, N_DEV)
        right = lax.rem(me + 1, N_DEV)

        _neighbor_barrier(left, right)

        out_ref[pl.ds(me * m, m), :] = x_ref[:, :]

        for h in range(N_DEV - 1):
            origin = lax.rem(me - h + N_DEV, N_DEV)
            sl = pl.ds(origin * m, m)
            rdma = pltpu.make_async_remote_copy(
                src_ref=out_ref.at[sl, :],
                dst_ref=out_ref.at[sl, :],
                send_sem=send_sems.at[h],
                recv_sem=recv_sems.at[h],
                device_id=(right,),
                device_id_type=pl.DeviceIdType.MESH,
            )
            rdma.start()
            rdma.wait()

        _exit_barrier(left, right)

    return pl.pallas_call(
        body,
        out_shape=jax.ShapeDtypeStruct((N_DEV * m, n), x_shard.dtype),
        in_specs=[pl.BlockSpec(memory_space=pltpu.VMEM)],
        out_specs=pl.BlockSpec(memory_space=pltpu.VMEM),
        scratch_shapes=[
            pltpu.SemaphoreType.DMA((N_DEV - 1,)),
            pltpu.SemaphoreType.DMA((N_DEV - 1,)),
        ],
        compiler_params=pltpu.CompilerParams(collective_id=0),
    )(x_shard)


def _ring_reduce_scatter(p):
    M, n = p.shape
    m = M // N_DEV

    def body(p_ref, out_ref, sbuf, rbuf, send_sems, recv_sems):
        me = lax.axis_index("i")
        left = lax.rem(me + N_DEV - 1, N_DEV)
        right = lax.rem(me + 1, N_DEV)

        _neighbor_barrier(left, right)

        for s in range(N_DEV - 1):
            o = lax.rem(me - 1 - s + 2 * N_DEV, N_DEV)
            blk = p_ref[pl.ds(o * m, m), :]
            if s == 0:
                sbuf[s, :, :] = blk
            else:
                sbuf[s, :, :] = blk + rbuf[s - 1, :, :]
            rdma = pltpu.make_async_remote_copy(
                src_ref=sbuf.at[s, :, :],
                dst_ref=rbuf.at[s, :, :],
                send_sem=send_sems.at[s],
                recv_sem=recv_sems.at[s],
                device_id=(right,),
                device_id_type=pl.DeviceIdType.MESH,
            )
            rdma.start()
            rdma.wait()

        out_ref[:, :] = p_ref[pl.ds(me * m, m), :] + rbuf[N_DEV - 2, :, :]

        _exit_barrier(left, right)

    return pl.pallas_call(
        body,
        out_shape=jax.ShapeDtypeStruct((m, n), p.dtype),
        in_specs=[pl.BlockSpec(memory_space=pltpu.VMEM)],
        out_specs=pl.BlockSpec(memory_space=pltpu.VMEM),
        scratch_shapes=[
            pltpu.VMEM((N_DEV - 1, m, n), p.dtype),
            pltpu.VMEM((N_DEV - 1, m, n), p.dtype),
            pltpu.SemaphoreType.DMA((N_DEV - 1,)),
            pltpu.SemaphoreType.DMA((N_DEV - 1,)),
        ],
        compiler_params=pltpu.CompilerParams(collective_id=1),
    )(p)


def kernel(x, W1, W2):
    x_full = _ring_allgather(x)
    h = x_full @ W1
    h = h * jax.nn.sigmoid(h)
    p = h @ W2
    return _ring_reduce_scatter(p)


# device time: 128664 ns/iter; 2.9953x vs baseline; 2.9953x over previous
import functools

import jax
import jax.numpy as jnp
from jax import lax
from jax.experimental import pallas as pl
from jax.experimental.pallas import tpu as pltpu

N_DEV = 4
F_TILE = 1024


def kernel(x, W1, W2):
    m2, d = x.shape
    m = m2 // 2
    f = W1.shape[1]

    xb = x.astype(jnp.bfloat16)
    W1b = W1.astype(jnp.bfloat16)
    W2b = W2.astype(jnp.bfloat16)

    def body(x_ref, w1_ref, w2_ref, out_ref,
             aga_recv, agb_recv, rsa_send, rsa_recv, rsb_send, rsb_recv,
             aga_ssem, aga_rsem, agb_ssem, agb_rsem,
             rsa_ssem, rsa_rsem, rsb_ssem, rsb_rsem):
        me = lax.axis_index("i")
        left = lax.rem(me + N_DEV - 1, N_DEV)
        right = lax.rem(me + 1, N_DEV)

        bar = pltpu.get_barrier_semaphore()
        for nbr in (left, right):
            pl.semaphore_signal(
                bar, inc=1,
                device_id=(nbr,), device_id_type=pl.DeviceIdType.MESH,
            )
        pl.semaphore_wait(bar, 2)

        def rc(src, dst, ssem, rsem, dev):
            return pltpu.make_async_remote_copy(
                src_ref=src, dst_ref=dst, send_sem=ssem, recv_sem=rsem,
                device_id=(dev,), device_id_type=pl.DeviceIdType.MESH,
            )

        def compute_p(xa):
            acc = None
            for ft in range(f // F_TILE):
                lo = ft * F_TILE
                h1 = jnp.dot(xa, w1_ref[:, lo:lo + F_TILE],
                             preferred_element_type=jnp.float32)
                h1 = h1 * jax.nn.sigmoid(h1)
                pt = jnp.dot(h1.astype(jnp.bfloat16),
                             w2_ref[lo:lo + F_TILE, :],
                             preferred_element_type=jnp.float32)
                acc = pt if acc is None else acc + pt
            return acc

        drain = []
        aga = [None] * (N_DEV - 1)
        agb = [None] * (N_DEV - 1)
        rsa = [None] * (N_DEV - 1)
        rsb = [None] * (N_DEV - 1)

        aga[0] = rc(x_ref.at[pl.ds(0, m), :], aga_recv.at[0],
                    aga_ssem.at[0], aga_rsem.at[0], right)
        agb[0] = rc(x_ref.at[pl.ds(m, m), :], agb_recv.at[0],
                    agb_ssem.at[0], agb_rsem.at[0], left)
        aga[0].start()
        agb[0].start()
        drain += [aga[0], agb[0]]

        out_ref[0:m, :] = compute_p(x_ref[0:m, :])
        out_ref[m:m2, :] = compute_p(x_ref[m:m2, :])

        for t in range(1, N_DEV):
            h = t - 1
            s = t - 1
            aga[h].wait_recv()
            agb[h].wait_recv()
            if t <= N_DEV - 2:
                aga[t] = rc(aga_recv.at[h], aga_recv.at[t],
                            aga_ssem.at[t], aga_rsem.at[t], right)
                agb[t] = rc(agb_recv.at[h], agb_recv.at[t],
                            agb_ssem.at[t], agb_rsem.at[t], left)
                aga[t].start()
                agb[t].start()
                drain += [aga[t], agb[t]]

            pa = compute_p(aga_recv[h])
            if s == 0:
                rsa_send[0, :, :] = pa.astype(jnp.bfloat16)
            else:
                rsa[s - 1].wait_recv()
                rsa_send[s, :, :] = (
                    pa + rsa_recv[s - 1, :, :].astype(jnp.float32)
                ).astype(jnp.bfloat16)
            rsa[s] = rc(rsa_send.at[s], rsa_recv.at[s],
                        rsa_ssem.at[s], rsa_rsem.at[s], right)
            rsa[s].start()
            drain.append(rsa[s])

            pb = compute_p(agb_recv[h])
            if s == 0:
                rsb_send[0, :, :] = pb.astype(jnp.bfloat16)
            else:
                rsb[s - 1].wait_recv()
                rsb_send[s, :, :] = (
                    pb + rsb_recv[s - 1, :, :].astype(jnp.float32)
                ).astype(jnp.bfloat16)
            rsb[s] = rc(rsb_send.at[s], rsb_recv.at[s],
                        rsb_ssem.at[s], rsb_rsem.at[s], left)
            rsb[s].start()
            drain.append(rsb[s])

        rsa[N_DEV - 2].wait_recv()
        out_ref[0:m, :] = (
            out_ref[0:m, :] + rsa_recv[N_DEV - 2, :, :].astype(jnp.float32)
        )
        rsb[N_DEV - 2].wait_recv()
        out_ref[m:m2, :] = (
            out_ref[m:m2, :] + rsb_recv[N_DEV - 2, :, :].astype(jnp.float32)
        )

        for r in drain:
            r.wait_send()

        @functools.partial(pl.run_scoped, sem=pltpu.SemaphoreType.REGULAR)
        def _(sem):
            for nbr in (left, right):
                pl.semaphore_signal(
                    sem, inc=1,
                    device_id=(nbr,), device_id_type=pl.DeviceIdType.MESH,
                )
            pl.semaphore_wait(sem, 2)

    return pl.pallas_call(
        body,
        out_shape=jax.ShapeDtypeStruct((m2, d), jnp.float32),
        in_specs=[pl.BlockSpec(memory_space=pltpu.VMEM)] * 3,
        out_specs=pl.BlockSpec(memory_space=pltpu.VMEM),
        scratch_shapes=[
            pltpu.VMEM((N_DEV - 1, m, d), jnp.bfloat16),
            pltpu.VMEM((N_DEV - 1, m, d), jnp.bfloat16),
            pltpu.VMEM((N_DEV - 1, m, d), jnp.bfloat16),
            pltpu.VMEM((N_DEV - 1, m, d), jnp.bfloat16),
            pltpu.VMEM((N_DEV - 1, m, d), jnp.bfloat16),
            pltpu.VMEM((N_DEV - 1, m, d), jnp.bfloat16),
        ] + [pltpu.SemaphoreType.DMA((N_DEV - 1,))] * 8,
        compiler_params=pltpu.CompilerParams(
            collective_id=0,
            vmem_limit_bytes=40 * 1024 * 1024,
        ),
    )(xb, W1b, W2b)


# device time: 128551 ns/iter; 2.9979x vs baseline; 1.0009x over previous
import functools

import jax
import jax.numpy as jnp
from jax import lax
from jax.experimental import pallas as pl
from jax.experimental.pallas import tpu as pltpu

N_DEV = 4
F_TILE = 2048


def kernel(x, W1, W2):
    m2, d = x.shape
    m = m2 // 2
    f = W1.shape[1]

    xb = x.astype(jnp.bfloat16)
    W1b = W1.astype(jnp.bfloat16)
    W2b = W2.astype(jnp.bfloat16)

    def body(x_ref, w1_ref, w2_ref, out_ref,
             aga_recv, agb_recv, rsa_send, rsa_recv, rsb_send, rsb_recv,
             aga_ssem, aga_rsem, agb_ssem, agb_rsem,
             rsa_ssem, rsa_rsem, rsb_ssem, rsb_rsem):
        me = lax.axis_index("i")
        left = lax.rem(me + N_DEV - 1, N_DEV)
        right = lax.rem(me + 1, N_DEV)

        bar = pltpu.get_barrier_semaphore()
        for nbr in (left, right):
            pl.semaphore_signal(
                bar, inc=1,
                device_id=(nbr,), device_id_type=pl.DeviceIdType.MESH,
            )
        pl.semaphore_wait(bar, 2)

        def rc(src, dst, ssem, rsem, dev):
            return pltpu.make_async_remote_copy(
                src_ref=src, dst_ref=dst, send_sem=ssem, recv_sem=rsem,
                device_id=(dev,), device_id_type=pl.DeviceIdType.MESH,
            )

        def compute_p(xa):
            acc = None
            for ft in range(f // F_TILE):
                lo = ft * F_TILE
                h1 = jnp.dot(xa, w1_ref[:, lo:lo + F_TILE],
                             preferred_element_type=jnp.float32)
                h1 = (h1 * jax.nn.sigmoid(h1)).astype(jnp.bfloat16)
                pt = jnp.dot(h1, w2_ref[lo:lo + F_TILE, :],
                             preferred_element_type=jnp.float32)
                acc = pt if acc is None else acc + pt
            return acc

        drain = []
        aga = [None] * (N_DEV - 1)
        agb = [None] * (N_DEV - 1)
        rsa = [None] * (N_DEV - 1)
        rsb = [None] * (N_DEV - 1)

        aga[0] = rc(x_ref.at[pl.ds(0, m), :], aga_recv.at[0],
                    aga_ssem.at[0], aga_rsem.at[0], right)
        agb[0] = rc(x_ref.at[pl.ds(m, m), :], agb_recv.at[0],
                    agb_ssem.at[0], agb_rsem.at[0], left)
        aga[0].start()
        agb[0].start()
        drain += [aga[0], agb[0]]

        out_ref[0:m, :] = compute_p(x_ref[0:m, :])
        out_ref[m:m2, :] = compute_p(x_ref[m:m2, :])

        for t in range(1, N_DEV):
            h = t - 1
            s = t - 1
            aga[h].wait_recv()
            agb[h].wait_recv()
            if t <= N_DEV - 2:
                aga[t] = rc(aga_recv.at[h], aga_recv.at[t],
                            aga_ssem.at[t], aga_rsem.at[t], right)
                agb[t] = rc(agb_recv.at[h], agb_recv.at[t],
                            agb_ssem.at[t], agb_rsem.at[t], left)
                aga[t].start()
                agb[t].start()
                drain += [aga[t], agb[t]]

            pa = compute_p(aga_recv[h])
            if s == 0:
                rsa_send[0, :, :] = pa.astype(jnp.bfloat16)
            else:
                rsa[s - 1].wait_recv()
                rsa_send[s, :, :] = (
                    pa + rsa_recv[s - 1, :, :].astype(jnp.float32)
                ).astype(jnp.bfloat16)
            rsa[s] = rc(rsa_send.at[s], rsa_recv.at[s],
                        rsa_ssem.at[s], rsa_rsem.at[s], right)
            rsa[s].start()
            drain.append(rsa[s])

            pb = compute_p(agb_recv[h])
            if s == 0:
                rsb_send[0, :, :] = pb.astype(jnp.bfloat16)
            else:
                rsb[s - 1].wait_recv()
                rsb_send[s, :, :] = (
                    pb + rsb_recv[s - 1, :, :].astype(jnp.float32)
                ).astype(jnp.bfloat16)
            rsb[s] = rc(rsb_send.at[s], rsb_recv.at[s],
                        rsb_ssem.at[s], rsb_rsem.at[s], left)
            rsb[s].start()
            drain.append(rsb[s])

        rsa[N_DEV - 2].wait_recv()
        out_ref[0:m, :] = (
            out_ref[0:m, :] + rsa_recv[N_DEV - 2, :, :].astype(jnp.float32)
        )
        rsb[N_DEV - 2].wait_recv()
        out_ref[m:m2, :] = (
            out_ref[m:m2, :] + rsb_recv[N_DEV - 2, :, :].astype(jnp.float32)
        )

        for r in drain:
            r.wait_send()

        @functools.partial(pl.run_scoped, sem=pltpu.SemaphoreType.REGULAR)
        def _(sem):
            for nbr in (left, right):
                pl.semaphore_signal(
                    sem, inc=1,
                    device_id=(nbr,), device_id_type=pl.DeviceIdType.MESH,
                )
            pl.semaphore_wait(sem, 2)

    return pl.pallas_call(
        body,
        out_shape=jax.ShapeDtypeStruct((m2, d), jnp.float32),
        in_specs=[pl.BlockSpec(memory_space=pltpu.VMEM)] * 3,
        out_specs=pl.BlockSpec(memory_space=pltpu.VMEM),
        scratch_shapes=[
            pltpu.VMEM((N_DEV - 1, m, d), jnp.bfloat16),
            pltpu.VMEM((N_DEV - 1, m, d), jnp.bfloat16),
            pltpu.VMEM((N_DEV - 1, m, d), jnp.bfloat16),
            pltpu.VMEM((N_DEV - 1, m, d), jnp.bfloat16),
            pltpu.VMEM((N_DEV - 1, m, d), jnp.bfloat16),
            pltpu.VMEM((N_DEV - 1, m, d), jnp.bfloat16),
        ] + [pltpu.SemaphoreType.DMA((N_DEV - 1,))] * 8,
        compiler_params=pltpu.CompilerParams(
            collective_id=0,
            vmem_limit_bytes=40 * 1024 * 1024,
        ),
    )(xb, W1b, W2b)


# device time: 118826 ns/iter; 3.2432x vs baseline; 1.0818x over previous
import functools

import jax
import jax.numpy as jnp
from jax import lax
from jax.experimental import pallas as pl
from jax.experimental.pallas import tpu as pltpu

N_DEV = 4
F_TILE = 2048


def kernel(x, W1, W2):
    m2, d = x.shape
    m = m2 // 2
    f = W1.shape[1]

    xb = x.astype(jnp.bfloat16)
    W1b = W1.astype(jnp.bfloat16)
    W2b = W2.astype(jnp.bfloat16)

    def body(x_ref, w1_ref, w2_ref, out_ref,
             aga_recv, agb_recv, rsa_send, rsa_recv, rsb_send, rsb_recv,
             aga_ssem, aga_rsem, agb_ssem, agb_rsem,
             rsa_ssem, rsa_rsem, rsb_ssem, rsb_rsem):
        me = lax.axis_index("i")
        left = lax.rem(me + N_DEV - 1, N_DEV)
        right = lax.rem(me + 1, N_DEV)

        bar = pltpu.get_barrier_semaphore()
        for nbr in (left, right):
            pl.semaphore_signal(
                bar, inc=1,
                device_id=(nbr,), device_id_type=pl.DeviceIdType.MESH,
            )
        pl.semaphore_wait(bar, 2)

        def rc(src, dst, ssem, rsem, dev):
            return pltpu.make_async_remote_copy(
                src_ref=src, dst_ref=dst, send_sem=ssem, recv_sem=rsem,
                device_id=(dev,), device_id_type=pl.DeviceIdType.MESH,
            )

        def compute_p(xa):
            acc = None
            for ft in range(f // F_TILE):
                lo = ft * F_TILE
                h1 = jnp.dot(xa, w1_ref[:, lo:lo + F_TILE],
                             preferred_element_type=jnp.float32
                             ).astype(jnp.bfloat16)
                h1 = h1 * jax.nn.sigmoid(h1)
                pt = jnp.dot(h1, w2_ref[lo:lo + F_TILE, :],
                             preferred_element_type=jnp.float32)
                acc = pt if acc is None else acc + pt
            return acc

        drain = []
        aga = [None] * (N_DEV - 1)
        agb = [None] * (N_DEV - 1)
        rsa = [None] * (N_DEV - 1)
        rsb = [None] * (N_DEV - 1)

        aga[0] = rc(x_ref.at[pl.ds(0, m), :], aga_recv.at[0],
                    aga_ssem.at[0], aga_rsem.at[0], right)
        agb[0] = rc(x_ref.at[pl.ds(m, m), :], agb_recv.at[0],
                    agb_ssem.at[0], agb_rsem.at[0], left)
        aga[0].start()
        agb[0].start()
        drain += [aga[0], agb[0]]

        out_ref[0:m, :] = compute_p(x_ref[0:m, :])

        for t in range(1, N_DEV):
            h = t - 1
            s = t - 1
            aga[h].wait_recv()
            agb[h].wait_recv()
            if t <= N_DEV - 2:
                aga[t] = rc(aga_recv.at[h], aga_recv.at[t],
                            aga_ssem.at[t], aga_rsem.at[t], right)
                agb[t] = rc(agb_recv.at[h], agb_recv.at[t],
                            agb_ssem.at[t], agb_rsem.at[t], left)
                aga[t].start()
                agb[t].start()
                drain += [aga[t], agb[t]]

            pa = compute_p(aga_recv[h])
            if s == 0:
                rsa_send[0, :, :] = pa.astype(jnp.bfloat16)
            else:
                rsa[s - 1].wait_recv()
                rsa_send[s, :, :] = (
                    pa + rsa_recv[s - 1, :, :].astype(jnp.float32)
                ).astype(jnp.bfloat16)
            rsa[s] = rc(rsa_send.at[s], rsa_recv.at[s],
                        rsa_ssem.at[s], rsa_rsem.at[s], right)
            rsa[s].start()
            drain.append(rsa[s])

            pb = compute_p(agb_recv[h])
            if s == 0:
                rsb_send[0, :, :] = pb.astype(jnp.bfloat16)
            else:
                rsb[s - 1].wait_recv()
                rsb_send[s, :, :] = (
                    pb + rsb_recv[s - 1, :, :].astype(jnp.float32)
                ).astype(jnp.bfloat16)
            rsb[s] = rc(rsb_send.at[s], rsb_recv.at[s],
                        rsb_ssem.at[s], rsb_rsem.at[s], left)
            rsb[s].start()
            drain.append(rsb[s])

        out_ref[m:m2, :] = compute_p(x_ref[m:m2, :])

        rsa[N_DEV - 2].wait_recv()
        out_ref[0:m, :] = (
            out_ref[0:m, :] + rsa_recv[N_DEV - 2, :, :].astype(jnp.float32)
        )
        rsb[N_DEV - 2].wait_recv()
        out_ref[m:m2, :] = (
            out_ref[m:m2, :] + rsb_recv[N_DEV - 2, :, :].astype(jnp.float32)
        )

        for r in drain:
            r.wait_send()

        @functools.partial(pl.run_scoped, sem=pltpu.SemaphoreType.REGULAR)
        def _(sem):
            for nbr in (left, right):
                pl.semaphore_signal(
                    sem, inc=1,
                    device_id=(nbr,), device_id_type=pl.DeviceIdType.MESH,
                )
            pl.semaphore_wait(sem, 2)

    return pl.pallas_call(
        body,
        out_shape=jax.ShapeDtypeStruct((m2, d), jnp.float32),
        in_specs=[pl.BlockSpec(memory_space=pltpu.VMEM)] * 3,
        out_specs=pl.BlockSpec(memory_space=pltpu.VMEM),
        scratch_shapes=[
            pltpu.VMEM((N_DEV - 1, m, d), jnp.bfloat16),
            pltpu.VMEM((N_DEV - 1, m, d), jnp.bfloat16),
            pltpu.VMEM((N_DEV - 1, m, d), jnp.bfloat16),
            pltpu.VMEM((N_DEV - 1, m, d), jnp.bfloat16),
            pltpu.VMEM((N_DEV - 1, m, d), jnp.bfloat16),
            pltpu.VMEM((N_DEV - 1, m, d), jnp.bfloat16),
        ] + [pltpu.SemaphoreType.DMA((N_DEV - 1,))] * 8,
        compiler_params=pltpu.CompilerParams(
            collective_id=0,
            vmem_limit_bytes=40 * 1024 * 1024,
        ),
    )(xb, W1b, W2b)


# device time: 114656 ns/iter; 3.3612x vs baseline; 1.0364x over previous
import functools

import jax
import jax.numpy as jnp
from jax import lax
from jax.experimental import pallas as pl
from jax.experimental.pallas import tpu as pltpu

N_DEV = 4
F_TILE = 1024


def kernel(x, W1, W2):
    m2, d = x.shape
    m = m2 // 2
    f = W1.shape[1]
    nt = f // F_TILE

    xb = x.astype(jnp.bfloat16)

    def body(x_ref, w1_hbm, w2_hbm, out_ref,
             w1b, w2b, wstage,
             aga_recv, agb_recv, rsa_send, rsa_recv, rsb_send, rsb_recv,
             wsem,
             aga_ssem, aga_rsem, agb_ssem, agb_rsem,
             rsa_ssem, rsa_rsem, rsb_ssem, rsb_rsem):
        me = lax.axis_index("i")
        left = lax.rem(me + N_DEV - 1, N_DEV)
        right = lax.rem(me + 1, N_DEV)

        bar = pltpu.get_barrier_semaphore()
        for nbr in (left, right):
            pl.semaphore_signal(
                bar, inc=1,
                device_id=(nbr,), device_id_type=pl.DeviceIdType.MESH,
            )
        pl.semaphore_wait(bar, 2)

        def rc(src, dst, ssem, rsem, dev):
            return pltpu.make_async_remote_copy(
                src_ref=src, dst_ref=dst, send_sem=ssem, recv_sem=rsem,
                device_id=(dev,), device_id_type=pl.DeviceIdType.MESH,
            )

        drain = []
        aga = [None] * (N_DEV - 1)
        agb = [None] * (N_DEV - 1)
        rsa = [None] * (N_DEV - 1)
        rsb = [None] * (N_DEV - 1)

        aga[0] = rc(x_ref.at[pl.ds(0, m), :], aga_recv.at[0],
                    aga_ssem.at[0], aga_rsem.at[0], right)
        agb[0] = rc(x_ref.at[pl.ds(m, m), :], agb_recv.at[0],
                    agb_ssem.at[0], agb_rsem.at[0], left)
        aga[0].start()
        agb[0].start()
        drain += [aga[0], agb[0]]

        def compute_p(xa):
            acc = None
            for ft in range(nt):
                lo = ft * F_TILE
                h1 = jnp.dot(xa, w1b[:, lo:lo + F_TILE],
                             preferred_element_type=jnp.float32
                             ).astype(jnp.bfloat16)
                h1 = h1 * jax.nn.sigmoid(h1)
                pt = jnp.dot(h1, w2b[lo:lo + F_TILE, :],
                             preferred_element_type=jnp.float32)
                acc = pt if acc is None else acc + pt
            return acc

        def w_dma(ft):
            lo = ft * F_TILE
            c1 = pltpu.make_async_copy(
                w1_hbm.at[:, pl.ds(lo, F_TILE)], wstage.at[0], wsem.at[0])
            c2 = pltpu.make_async_copy(
                w2_hbm.at[pl.ds(lo, F_TILE), :], wstage.at[1], wsem.at[1])
            c1.start()
            c2.start()
            return c1, c2

        xa0 = x_ref[0:m, :]
        acc0 = None
        cur = w_dma(0)
        for ft in range(nt):
            lo = ft * F_TILE
            cur[0].wait()
            w1b[:, lo:lo + F_TILE] = wstage[0, :, :].astype(jnp.bfloat16)
            cur[1].wait()
            w2b[lo:lo + F_TILE, :] = (
                wstage[1, :, :].astype(jnp.bfloat16))
            if ft + 1 < nt:
                cur = w_dma(ft + 1)
            h1 = jnp.dot(xa0, w1b[:, lo:lo + F_TILE],
                         preferred_element_type=jnp.float32
                         ).astype(jnp.bfloat16)
            h1 = h1 * jax.nn.sigmoid(h1)
            pt = jnp.dot(h1, w2b[lo:lo + F_TILE, :],
                         preferred_element_type=jnp.float32)
            acc0 = pt if acc0 is None else acc0 + pt
        out_ref[0:m, :] = acc0

        for t in range(1, N_DEV):
            h = t - 1
            s = t - 1
            aga[h].wait_recv()
            agb[h].wait_recv()
            if t <= N_DEV - 2:
                aga[t] = rc(aga_recv.at[h], aga_recv.at[t],
                            aga_ssem.at[t], aga_rsem.at[t], right)
                agb[t] = rc(agb_recv.at[h], agb_recv.at[t],
                            agb_ssem.at[t], agb_rsem.at[t], left)
                aga[t].start()
                agb[t].start()
                drain += [aga[t], agb[t]]

            pa = compute_p(aga_recv[h])
            if s == 0:
                rsa_send[0, :, :] = pa.astype(jnp.bfloat16)
            else:
                rsa[s - 1].wait_recv()
                rsa_send[s, :, :] = (
                    pa + rsa_recv[s - 1, :, :].astype(jnp.float32)
                ).astype(jnp.bfloat16)
            rsa[s] = rc(rsa_send.at[s], rsa_recv.at[s],
                        rsa_ssem.at[s], rsa_rsem.at[s], right)
            rsa[s].start()
            drain.append(rsa[s])

            pb = compute_p(agb_recv[h])
            if s == 0:
                rsb_send[0, :, :] = pb.astype(jnp.bfloat16)
            else:
                rsb[s - 1].wait_recv()
                rsb_send[s, :, :] = (
                    pb + rsb_recv[s - 1, :, :].astype(jnp.float32)
                ).astype(jnp.bfloat16)
            rsb[s] = rc(rsb_send.at[s], rsb_recv.at[s],
                        rsb_ssem.at[s], rsb_rsem.at[s], left)
            rsb[s].start()
            drain.append(rsb[s])

        out_ref[m:m2, :] = compute_p(x_ref[m:m2, :])

        rsa[N_DEV - 2].wait_recv()
        out_ref[0:m, :] = (
            out_ref[0:m, :] + rsa_recv[N_DEV - 2, :, :].astype(jnp.float32)
        )
        rsb[N_DEV - 2].wait_recv()
        out_ref[m:m2, :] = (
            out_ref[m:m2, :] + rsb_recv[N_DEV - 2, :, :].astype(jnp.float32)
        )

        for r in drain:
            r.wait_send()

        @functools.partial(pl.run_scoped, sem=pltpu.SemaphoreType.REGULAR)
        def _(sem):
            for nbr in (left, right):
                pl.semaphore_signal(
                    sem, inc=1,
                    device_id=(nbr,), device_id_type=pl.DeviceIdType.MESH,
                )
            pl.semaphore_wait(sem, 2)

    return pl.pallas_call(
        body,
        out_shape=jax.ShapeDtypeStruct((m2, d), jnp.float32),
        in_specs=[
            pl.BlockSpec(memory_space=pltpu.VMEM),
            pl.BlockSpec(memory_space=pltpu.MemorySpace.HBM),
            pl.BlockSpec(memory_space=pltpu.MemorySpace.HBM),
        ],
        out_specs=pl.BlockSpec(memory_space=pltpu.VMEM),
        scratch_shapes=[
            pltpu.VMEM((d, f), jnp.bfloat16),
            pltpu.VMEM((f, d), jnp.bfloat16),
            pltpu.VMEM((2, d, F_TILE), jnp.float32),
            pltpu.VMEM((N_DEV - 1, m, d), jnp.bfloat16),
            pltpu.VMEM((N_DEV - 1, m, d), jnp.bfloat16),
            pltpu.VMEM((N_DEV - 1, m, d), jnp.bfloat16),
            pltpu.VMEM((N_DEV - 1, m, d), jnp.bfloat16),
            pltpu.VMEM((N_DEV - 1, m, d), jnp.bfloat16),
            pltpu.VMEM((N_DEV - 1, m, d), jnp.bfloat16),
            pltpu.SemaphoreType.DMA((2,)),
        ] + [pltpu.SemaphoreType.DMA((N_DEV - 1,))] * 8,
        compiler_params=pltpu.CompilerParams(
            collective_id=0,
            vmem_limit_bytes=52 * 1024 * 1024,
        ),
    )(xb, W1, W2)


# device time: 106721 ns/iter; 3.6111x vs baseline; 1.0744x over previous
import functools

import jax
import jax.numpy as jnp
from jax import lax
from jax.experimental import pallas as pl
from jax.experimental.pallas import tpu as pltpu

N_DEV = 4
F_TILE = 1024
F_STREAM = 512


def kernel(x, W1, W2):
    m2, d = x.shape
    m = m2 // 2
    f = W1.shape[1]
    nt = f // F_TILE

    xb = x.astype(jnp.bfloat16)

    def body(x_ref, w1_hbm, w2_hbm, out_ref,
             w1b, w2b, w1stage, w2stage,
             aga_recv, agb_recv, rsa_send, rsa_recv, rsb_send, rsb_recv,
             wsem,
             aga_ssem, aga_rsem, agb_ssem, agb_rsem,
             rsa_ssem, rsa_rsem, rsb_ssem, rsb_rsem):
        me = lax.axis_index("i")
        left = lax.rem(me + N_DEV - 1, N_DEV)
        right = lax.rem(me + 1, N_DEV)

        bar = pltpu.get_barrier_semaphore()
        for nbr in (left, right):
            pl.semaphore_signal(
                bar, inc=1,
                device_id=(nbr,), device_id_type=pl.DeviceIdType.MESH,
            )
        pl.semaphore_wait(bar, 2)

        def rc(src, dst, ssem, rsem, dev):
            return pltpu.make_async_remote_copy(
                src_ref=src, dst_ref=dst, send_sem=ssem, recv_sem=rsem,
                device_id=(dev,), device_id_type=pl.DeviceIdType.MESH,
            )

        drain = []
        aga = [None] * (N_DEV - 1)
        agb = [None] * (N_DEV - 1)
        rsa = [None] * (N_DEV - 1)
        rsb = [None] * (N_DEV - 1)

        aga[0] = rc(x_ref.at[pl.ds(0, m), :], aga_recv.at[0],
                    aga_ssem.at[0], aga_rsem.at[0], right)
        agb[0] = rc(x_ref.at[pl.ds(m, m), :], agb_recv.at[0],
                    agb_ssem.at[0], agb_rsem.at[0], left)
        aga[0].start()
        agb[0].start()
        drain += [aga[0], agb[0]]

        def compute_p(xa):
            acc = None
            for ft in range(nt):
                lo = ft * F_TILE
                h1 = jnp.dot(xa, w1b[:, lo:lo + F_TILE],
                             preferred_element_type=jnp.float32
                             ).astype(jnp.bfloat16)
                h1 = h1 * jax.nn.sigmoid(h1)
                pt = jnp.dot(h1, w2b[lo:lo + F_TILE, :],
                             preferred_element_type=jnp.float32)
                acc = pt if acc is None else acc + pt
            return acc

        ns = f // F_STREAM

        def w_dma(ft):
            lo = ft * F_STREAM
            b = ft % 2
            c1 = pltpu.make_async_copy(
                w1_hbm.at[:, pl.ds(lo, F_STREAM)], w1stage.at[b],
                wsem.at[b])
            c2 = pltpu.make_async_copy(
                w2_hbm.at[pl.ds(lo, F_STREAM), :], w2stage.at[b],
                wsem.at[2 + b])
            c1.start()
            c2.start()
            return c1, c2

        xa0 = x_ref[0:m, :]
        acc0 = None
        pend = [w_dma(0), w_dma(1)]
        for ft in range(ns):
            b = ft % 2
            lo = ft * F_STREAM
            c1, c2 = pend[b]
            c1.wait()
            w1b[:, lo:lo + F_STREAM] = w1stage[b, :, :].astype(jnp.bfloat16)
            c2.wait()
            w2b[lo:lo + F_STREAM, :] = w2stage[b, :, :].astype(jnp.bfloat16)
            if ft + 2 < ns:
                pend[b] = w_dma(ft + 2)
            h1 = jnp.dot(xa0, w1b[:, lo:lo + F_STREAM],
                         preferred_element_type=jnp.float32
                         ).astype(jnp.bfloat16)
            h1 = h1 * jax.nn.sigmoid(h1)
            pt = jnp.dot(h1, w2b[lo:lo + F_STREAM, :],
                         preferred_element_type=jnp.float32)
            acc0 = pt if acc0 is None else acc0 + pt
        out_ref[0:m, :] = acc0

        for t in range(1, N_DEV):
            h = t - 1
            s = t - 1
            aga[h].wait_recv()
            agb[h].wait_recv()
            if t <= N_DEV - 2:
                aga[t] = rc(aga_recv.at[h], aga_recv.at[t],
                            aga_ssem.at[t], aga_rsem.at[t], right)
                agb[t] = rc(agb_recv.at[h], agb_recv.at[t],
                            agb_ssem.at[t], agb_rsem.at[t], left)
                aga[t].start()
                agb[t].start()
                drain += [aga[t], agb[t]]

            pa = compute_p(aga_recv[h])
            if s == 0:
                rsa_send[0, :, :] = pa.astype(jnp.bfloat16)
            else:
                rsa[s - 1].wait_recv()
                rsa_send[s, :, :] = (
                    pa + rsa_recv[s - 1, :, :].astype(jnp.float32)
                ).astype(jnp.bfloat16)
            rsa[s] = rc(rsa_send.at[s], rsa_recv.at[s],
                        rsa_ssem.at[s], rsa_rsem.at[s], right)
            rsa[s].start()
            drain.append(rsa[s])

            pb = compute_p(agb_recv[h])
            if s == 0:
                rsb_send[0, :, :] = pb.astype(jnp.bfloat16)
            else:
                rsb[s - 1].wait_recv()
                rsb_send[s, :, :] = (
                    pb + rsb_recv[s - 1, :, :].astype(jnp.float32)
                ).astype(jnp.bfloat16)
            rsb[s] = rc(rsb_send.at[s], rsb_recv.at[s],
                        rsb_ssem.at[s], rsb_rsem.at[s], left)
            rsb[s].start()
            drain.append(rsb[s])

        out_ref[m:m2, :] = compute_p(x_ref[m:m2, :])

        rsa[N_DEV - 2].wait_recv()
        out_ref[0:m, :] = (
            out_ref[0:m, :] + rsa_recv[N_DEV - 2, :, :].astype(jnp.float32)
        )
        rsb[N_DEV - 2].wait_recv()
        out_ref[m:m2, :] = (
            out_ref[m:m2, :] + rsb_recv[N_DEV - 2, :, :].astype(jnp.float32)
        )

        for r in drain:
            r.wait_send()

        @functools.partial(pl.run_scoped, sem=pltpu.SemaphoreType.REGULAR)
        def _(sem):
            for nbr in (left, right):
                pl.semaphore_signal(
                    sem, inc=1,
                    device_id=(nbr,), device_id_type=pl.DeviceIdType.MESH,
                )
            pl.semaphore_wait(sem, 2)

    return pl.pallas_call(
        body,
        out_shape=jax.ShapeDtypeStruct((m2, d), jnp.float32),
        in_specs=[
            pl.BlockSpec(memory_space=pltpu.VMEM),
            pl.BlockSpec(memory_space=pltpu.MemorySpace.HBM),
            pl.BlockSpec(memory_space=pltpu.MemorySpace.HBM),
        ],
        out_specs=pl.BlockSpec(memory_space=pltpu.VMEM),
        scratch_shapes=[
            pltpu.VMEM((d, f), jnp.bfloat16),
            pltpu.VMEM((f, d), jnp.bfloat16),
            pltpu.VMEM((2, d, F_STREAM), jnp.float32),
            pltpu.VMEM((2, F_STREAM, d), jnp.float32),
            pltpu.VMEM((N_DEV - 1, m, d), jnp.bfloat16),
            pltpu.VMEM((N_DEV - 1, m, d), jnp.bfloat16),
            pltpu.VMEM((N_DEV - 1, m, d), jnp.bfloat16),
            pltpu.VMEM((N_DEV - 1, m, d), jnp.bfloat16),
            pltpu.VMEM((N_DEV - 1, m, d), jnp.bfloat16),
            pltpu.VMEM((N_DEV - 1, m, d), jnp.bfloat16),
            pltpu.SemaphoreType.DMA((4,)),
        ] + [pltpu.SemaphoreType.DMA((N_DEV - 1,))] * 8,
        compiler_params=pltpu.CompilerParams(
            collective_id=0,
            vmem_limit_bytes=56 * 1024 * 1024,
        ),
    )(xb, W1, W2)
